# baseline (speedup 1.0000x reference)
import numpy as np
import jax
import jax.numpy as jnp
from jax.experimental import pallas as pl

B = 1024
SEQ = 10
HW = 16
NTAB = 19
NSZ = 7
C = 24
WC = HW * C

BS = 8
F = BS * SEQ
R = F * HW

_INV_SQRT2 = float(1.0 / np.sqrt(2.0))

_SEL = np.zeros((HW, 3, HW), np.float32)
for _wi in range(HW):
    for _t in range(3):
        _w = _wi - _t + 1
        if 0 <= _w < HW:
            _SEL[_wi, _t, _w] = 1.0


def _dot(a, b):
    return jax.lax.dot_general(a, b, (((1,), (0,)), ((), ())),
                               preferred_element_type=jnp.float32)


def _gelu(v):
    return v * 0.5 * (1.0 + jax.lax.erf(v * _INV_SQRT2))


def _shift_rows(a, off):
    n, c = a.shape
    if off > 0:
        return jnp.concatenate(
            [a[off:], jnp.zeros((off, c), jnp.float32)], axis=0)
    if off < 0:
        return jnp.concatenate(
            [jnp.zeros((-off, c), jnp.float32), a[:off]], axis=0)
    return a


def _softmax(x):
    m = jnp.max(x, axis=1, keepdims=True)
    e = jnp.exp(x - m)
    return e / jnp.sum(e, axis=1, keepdims=True)


def _conv_stack(a, m_neg, m_pos):
    return jnp.concatenate(
        [m_neg * _shift_rows(a, -1), a, m_pos * _shift_rows(a, 1)], axis=1)


def _encoder(x2, m_neg, m_pos, a1, b1, a2, b2, q, fcb):
    r1 = jax.nn.relu(_dot(_conv_stack(x2, m_neg, m_pos), a1) + b1)
    r2 = jax.nn.relu(_dot(_conv_stack(r1, m_neg, m_pos), a2) + b2)
    t = _dot(r2, q)
    return t.reshape(F, HW, C).sum(axis=1) + fcb


def _body(refs, n_params):
    raw_ref, norm_ref, tab_ref = refs[0], refs[1], refs[2]
    c = [r[...] for r in refs[3:3 + n_params]]
    ol, oo, org, oprob = refs[3 + n_params:]

    it = iter(c)

    def nxt():
        return next(it)

    enc_w = [(nxt(), nxt(), nxt(), nxt(), nxt(), nxt()) for _ in range(3)]
    ti_w = nxt()
    bn_sc, bn_b = nxt(), nxt()
    blk = [[(nxt(), nxt()) for _ in range(3)] for _ in range(3)]
    gp_w, gp_b = nxt(), nxt()
    tr_w, tr_b = nxt(), nxt()
    ln_g, ln_b = nxt(), nxt()
    tb1_w, tb1_b = nxt(), nxt()
    tb2_w, tb2_b = nxt(), nxt()
    t2t_w, t2t_b = nxt(), nxt()
    fus_w, fus_b = nxt(), nxt()
    sc1_w, sc1_b = nxt(), nxt()
    sc2_w, sc2_b = nxt(), nxt()
    so1_w, so1_b = nxt(), nxt()
    so2_w, so2_b = nxt(), nxt()
    sr1_w, sr1_b = nxt(), nxt()
    sr2_w, sr2_b = nxt(), nxt()

    raw = raw_ref[...]
    norm = norm_ref[...]
    tab = tab_ref[...]

    delta = jnp.concatenate(
        [jnp.zeros_like(norm[:, :1]), norm[:, 1:] - norm[:, :-1]], axis=1)

    hrow = jax.lax.broadcasted_iota(jnp.int32, (R, 1), 0) % HW
    m_neg = (hrow >= 1).astype(jnp.float32)
    m_pos = (hrow <= HW - 2).astype(jnp.float32)

    enc_outs = []
    for x4, wset in zip((raw, norm, delta), enc_w):
        x2 = x4.reshape(R, HW)
        enc_outs.append(_encoder(x2, m_neg, m_pos, *wset))

    seq = jnp.concatenate(enc_outs, axis=1)
    s = jax.nn.relu(_dot(seq, ti_w) * bn_sc + bn_b)
    s3 = s.reshape(BS, SEQ, 48)

    zpad = {d: jnp.zeros((BS, d, 48), jnp.float32) for d in (1, 2, 4)}
    for i in range(3):
        acc = None
        for di, d in enumerate((1, 2, 4)):
            w, b = blk[i][di]
            prev = jnp.concatenate([zpad[d], s3[:, :SEQ - d]], axis=1)
            nxt_ = jnp.concatenate([s3[:, d:], zpad[d]], axis=1)
            y = (_dot(prev.reshape(F, 48), w[0]) +
                 _dot(s3.reshape(F, 48), w[1]) +
                 _dot(nxt_.reshape(F, 48), w[2]) + b)
            acc = y if acc is None else acc + y
        s3 = jax.nn.relu(s3 + acc.reshape(BS, SEQ, 48))

    scores = jnp.sum(s3 * gp_w, axis=2) + gp_b
    attn = _softmax(scores)

    amp = raw.sum(axis=3).sum(axis=2) * np.float32(1.0 / (HW * HW))
    thr = amp.mean(axis=1, keepdims=True)
    m_hi = (amp >= thr).astype(jnp.float32)
    m_lo = 1.0 - m_hi

    gfeat = None
    ph = None
    plo = None
    for t in range(SEQ):
        st = s3[:, t]
        ga = attn[:, t:t + 1] * st
        ha = m_hi[:, t:t + 1] * st
        la = m_lo[:, t:t + 1] * st
        gfeat = ga if gfeat is None else gfeat + ga
        ph = ha if ph is None else ph + ha
        plo = la if plo is None else plo + la
    ph = ph / (jnp.sum(m_hi, axis=1, keepdims=True) + 1e-6)
    plo = plo / (jnp.sum(m_lo, axis=1, keepdims=True) + 1e-6)

    fused = jnp.concatenate([gfeat, ph, plo], axis=1)
    trunk = jax.nn.relu(_dot(fused, tr_w) + tr_b)

    mu = tab.mean(axis=1, keepdims=True)
    xc = tab - mu
    var = (xc * xc).mean(axis=1, keepdims=True)
    tn = xc / jnp.sqrt(var + 1e-5) * ln_g + ln_b
    tf = _gelu(_dot(tn, tb1_w) + tb1_b)
    tf = _gelu(_dot(tf, tb2_w) + tb2_b)
    tp = _dot(tf, t2t_w) + t2t_b

    hyb = jax.nn.relu(
        _dot(jnp.concatenate([trunk, tp, trunk * tp], axis=1), fus_w) + fus_b)
    sl = _dot(jax.nn.relu(_dot(hyb, sc1_w) + sc1_b), sc2_w) + sc2_b
    so = _dot(jax.nn.relu(_dot(hyb, so1_w) + so1_b), so2_w) + so2_b
    sp = _softmax(sl)
    vals = (jax.lax.broadcasted_iota(jnp.int32, (1, NSZ), 1).astype(
        jnp.float32) * np.float32(1.0 / (NSZ - 1)))
    expected = jnp.sum(sp * vals, axis=1, keepdims=True)
    res = 0.35 * jnp.tanh(
        _dot(jax.nn.relu(_dot(hyb, sr1_w) + sr1_b), sr2_w) + sr2_b)
    rg = jnp.clip(expected + res, 0.0, 1.0)

    ol[...] = sl
    oo[...] = so
    org[...] = rg
    oprob[...] = sp


def _prep_params(p):
    sel = jnp.asarray(_SEL)
    out = []
    for enc in ('amp', 'shp', 'dlt'):
        w1 = p[enc + '_c1w'].reshape(3, 3, C)
        a1 = jnp.einsum('itw,otc->oiwc', sel, w1).reshape(3 * HW, WC)
        b1 = jnp.tile(p[enc + '_c1b'], HW).reshape(1, WC)
        w2 = p[enc + '_c2w']
        a2 = jnp.einsum('itw,otcd->oicwd', sel, w2).reshape(3 * WC, WC)
        b2 = jnp.tile(p[enc + '_c2b'], HW).reshape(1, WC)
        q = jnp.tile(p[enc + '_fcw'] * np.float32(1.0 / (HW * HW)), (HW, 1))
        fcb = p[enc + '_fcb'].reshape(1, C)
        out += [a1, b1, a2, b2, q, fcb]
    out += [p['ti_w'].reshape(72, 48),
            p['bn_g'].reshape(1, 48) * np.float32(1.0 / np.sqrt(1.0 + 1e-5)),
            p['bn_b'].reshape(1, 48)]
    for i in range(3):
        for d in (1, 2, 4):
            out += [p[f'blk{i}_d{d}_w'], p[f'blk{i}_d{d}_b'].reshape(1, 48)]
    out += [p['gp_w'].reshape(1, 1, 48), p['gp_b'].reshape(1, 1),
            p['tr_w'], p['tr_b'].reshape(1, 64),
            p['ln_g'].reshape(1, NTAB), p['ln_b'].reshape(1, NTAB),
            p['tb1_w'], p['tb1_b'].reshape(1, 64),
            p['tb2_w'], p['tb2_b'].reshape(1, 64),
            p['t2t_w'], p['t2t_b'].reshape(1, 64),
            p['fus_w'], p['fus_b'].reshape(1, 96),
            p['sc1_w'], p['sc1_b'].reshape(1, 96),
            p['sc2_w'], p['sc2_b'].reshape(1, NSZ),
            p['so1_w'], p['so1_b'].reshape(1, 48),
            p['so2_w'], p['so2_b'].reshape(1, 6),
            p['sr1_w'], p['sr1_b'].reshape(1, 96),
            p['sr2_w'], p['sr2_b'].reshape(1, 1)]
    return out


def kernel(raw_window, norm_window, tabular_x, params):
    consts = _prep_params(params)
    n_params = len(consts)

    grid = B // BS
    in_specs = [
        pl.BlockSpec((BS, SEQ, HW, HW), lambda i: (i, 0, 0, 0)),
        pl.BlockSpec((BS, SEQ, HW, HW), lambda i: (i, 0, 0, 0)),
        pl.BlockSpec((BS, NTAB), lambda i: (i, 0)),
    ]
    for a in consts:
        nd = a.ndim
        in_specs.append(
            pl.BlockSpec(a.shape, lambda i, _nd=nd: (0,) * _nd))

    out_shape = [
        jax.ShapeDtypeStruct((B, NSZ), jnp.float32),
        jax.ShapeDtypeStruct((B, 6), jnp.float32),
        jax.ShapeDtypeStruct((B, 1), jnp.float32),
        jax.ShapeDtypeStruct((B, NSZ), jnp.float32),
    ]
    out_specs = [
        pl.BlockSpec((BS, NSZ), lambda i: (i, 0)),
        pl.BlockSpec((BS, 6), lambda i: (i, 0)),
        pl.BlockSpec((BS, 1), lambda i: (i, 0)),
        pl.BlockSpec((BS, NSZ), lambda i: (i, 0)),
    ]

    fn = pl.pallas_call(
        lambda *refs: _body(refs, n_params),
        grid=(grid,),
        in_specs=in_specs,
        out_specs=out_specs,
        out_shape=out_shape,
    )
    return tuple(fn(raw_window, norm_window, tabular_x, *consts))


# speedup vs baseline: 1.1783x; 1.1783x over previous
import numpy as np
import jax
import jax.numpy as jnp
from jax.experimental import pallas as pl

B = 1024
SEQ = 10
HW = 16
NTAB = 19
NSZ = 7
C = 24
WC = HW * C

BS = 16
F = BS * SEQ
R = F * HW

BS2 = 128
F2 = BS2 * SEQ

_INV_SQRT2 = float(1.0 / np.sqrt(2.0))

_SEL = np.zeros((HW, 3, HW), np.float32)
for _wi in range(HW):
    for _t in range(3):
        _w = _wi - _t + 1
        if 0 <= _w < HW:
            _SEL[_wi, _t, _w] = 1.0


def _dot(a, b):
    return jax.lax.dot_general(a, b, (((1,), (0,)), ((), ())),
                               preferred_element_type=jnp.float32)


def _gelu(v):
    return v * 0.5 * (1.0 + jax.lax.erf(v * _INV_SQRT2))


def _shift_rows(a, off):
    n, c = a.shape
    if off > 0:
        return jnp.concatenate(
            [a[off:], jnp.zeros((off, c), a.dtype)], axis=0)
    if off < 0:
        return jnp.concatenate(
            [jnp.zeros((-off, c), a.dtype), a[:off]], axis=0)
    return a


def _softmax(x):
    m = jnp.max(x, axis=1, keepdims=True)
    e = jnp.exp(x - m)
    return e / jnp.sum(e, axis=1, keepdims=True)


def _conv_stack(a, m_neg, m_pos):
    return jnp.concatenate(
        [m_neg * _shift_rows(a, -1), a, m_pos * _shift_rows(a, 1)], axis=1)


def _encoder(x2b, m_neg, m_pos, a1, b1, a2, b2, q, fcb):
    r1 = jax.nn.relu(_dot(_conv_stack(x2b, m_neg, m_pos), a1) + b1)
    r1b = r1.astype(jnp.bfloat16)
    r2 = jax.nn.relu(_dot(_conv_stack(r1b, m_neg, m_pos), a2) + b2)
    t = _dot(r2.astype(jnp.bfloat16), q)
    return t.reshape(F, HW, C).sum(axis=1) + fcb


def _enc_body(*refs):
    raw_ref, norm_ref = refs[0], refs[1]
    cw = [r[...] for r in refs[2:20]]
    feat_out, amp_out = refs[20], refs[21]

    raw = raw_ref[...]
    norm = norm_ref[...]
    delta = jnp.concatenate(
        [jnp.zeros_like(norm[:, :1]), norm[:, 1:] - norm[:, :-1]], axis=1)

    hrow = jax.lax.broadcasted_iota(jnp.int32, (R, 1), 0) % HW
    m_neg = (hrow >= 1).astype(jnp.bfloat16)
    m_pos = (hrow <= HW - 2).astype(jnp.bfloat16)

    outs = []
    for k, x4 in enumerate((raw, norm, delta)):
        x2b = x4.reshape(R, HW).astype(jnp.bfloat16)
        outs.append(_encoder(x2b, m_neg, m_pos, *cw[6 * k:6 * k + 6]))
    feat_out[...] = jnp.concatenate(outs, axis=1)
    amp_out[...] = raw.sum(axis=3).sum(axis=2) * np.float32(1.0 / (HW * HW))


def _seq_body(refs, n_params):
    feat_ref, amp_ref, tab_ref = refs[0], refs[1], refs[2]
    c = [r[...] for r in refs[3:3 + n_params]]
    ol, oo, org, oprob = refs[3 + n_params:]

    it = iter(c)

    def nxt():
        return next(it)

    ti_w = nxt()
    bn_sc, bn_b = nxt(), nxt()
    blk = [[(nxt(), nxt()) for _ in range(3)] for _ in range(3)]
    gp_w, gp_b = nxt(), nxt()
    tr_w, tr_b = nxt(), nxt()
    ln_g, ln_b = nxt(), nxt()
    tb1_w, tb1_b = nxt(), nxt()
    tb2_w, tb2_b = nxt(), nxt()
    t2t_w, t2t_b = nxt(), nxt()
    fus_w, fus_b = nxt(), nxt()
    sc1_w, sc1_b = nxt(), nxt()
    sc2_w, sc2_b = nxt(), nxt()
    so1_w, so1_b = nxt(), nxt()
    so2_w, so2_b = nxt(), nxt()
    sr1_w, sr1_b = nxt(), nxt()
    sr2_w, sr2_b = nxt(), nxt()

    feat = feat_ref[...]
    amp = amp_ref[...]
    tab = tab_ref[...]

    s = jax.nn.relu(_dot(feat, ti_w) * bn_sc + bn_b)
    s3 = s.reshape(BS2, SEQ, 48)

    zpad = {d: jnp.zeros((BS2, d, 48), jnp.float32) for d in (1, 2, 4)}
    for i in range(3):
        acc = None
        for di, d in enumerate((1, 2, 4)):
            w, b = blk[i][di]
            prev = jnp.concatenate([zpad[d], s3[:, :SEQ - d]], axis=1)
            nxt_ = jnp.concatenate([s3[:, d:], zpad[d]], axis=1)
            y = (_dot(prev.reshape(F2, 48), w[0]) +
                 _dot(s3.reshape(F2, 48), w[1]) +
                 _dot(nxt_.reshape(F2, 48), w[2]) + b)
            acc = y if acc is None else acc + y
        s3 = jax.nn.relu(s3 + acc.reshape(BS2, SEQ, 48))

    scores = jnp.sum(s3 * gp_w, axis=2) + gp_b
    attn = _softmax(scores)

    thr = amp.mean(axis=1, keepdims=True)
    m_hi = (amp >= thr).astype(jnp.float32)
    m_lo = 1.0 - m_hi

    gfeat = None
    ph = None
    plo = None
    for t in range(SEQ):
        st = s3[:, t]
        ga = attn[:, t:t + 1] * st
        ha = m_hi[:, t:t + 1] * st
        la = m_lo[:, t:t + 1] * st
        gfeat = ga if gfeat is None else gfeat + ga
        ph = ha if ph is None else ph + ha
        plo = la if plo is None else plo + la
    ph = ph / (jnp.sum(m_hi, axis=1, keepdims=True) + 1e-6)
    plo = plo / (jnp.sum(m_lo, axis=1, keepdims=True) + 1e-6)

    fused = jnp.concatenate([gfeat, ph, plo], axis=1)
    trunk = jax.nn.relu(_dot(fused, tr_w) + tr_b)

    mu = tab.mean(axis=1, keepdims=True)
    xc = tab - mu
    var = (xc * xc).mean(axis=1, keepdims=True)
    tn = xc / jnp.sqrt(var + 1e-5) * ln_g + ln_b
    tf = _gelu(_dot(tn, tb1_w) + tb1_b)
    tf = _gelu(_dot(tf, tb2_w) + tb2_b)
    tp = _dot(tf, t2t_w) + t2t_b

    hyb = jax.nn.relu(
        _dot(jnp.concatenate([trunk, tp, trunk * tp], axis=1), fus_w) + fus_b)
    sl = _dot(jax.nn.relu(_dot(hyb, sc1_w) + sc1_b), sc2_w) + sc2_b
    so = _dot(jax.nn.relu(_dot(hyb, so1_w) + so1_b), so2_w) + so2_b
    sp = _softmax(sl)
    vals = (jax.lax.broadcasted_iota(jnp.int32, (1, NSZ), 1).astype(
        jnp.float32) * np.float32(1.0 / (NSZ - 1)))
    expected = jnp.sum(sp * vals, axis=1, keepdims=True)
    res = 0.35 * jnp.tanh(
        _dot(jax.nn.relu(_dot(hyb, sr1_w) + sr1_b), sr2_w) + sr2_b)
    rg = jnp.clip(expected + res, 0.0, 1.0)

    ol[...] = sl
    oo[...] = so
    org[...] = rg
    oprob[...] = sp


def _prep_enc_params(p):
    sel = jnp.asarray(_SEL)
    out = []
    for enc in ('amp', 'shp', 'dlt'):
        w1 = p[enc + '_c1w'].reshape(3, 3, C)
        a1 = jnp.einsum('itw,otc->oiwc', sel, w1).reshape(3 * HW, WC)
        b1 = jnp.tile(p[enc + '_c1b'], HW).reshape(1, WC)
        w2 = p[enc + '_c2w']
        a2 = jnp.einsum('itw,otcd->oicwd', sel, w2).reshape(3 * WC, WC)
        b2 = jnp.tile(p[enc + '_c2b'], HW).reshape(1, WC)
        q = jnp.tile(p[enc + '_fcw'] * np.float32(1.0 / (HW * HW)), (HW, 1))
        fcb = p[enc + '_fcb'].reshape(1, C)
        out += [a1.astype(jnp.bfloat16), b1, a2.astype(jnp.bfloat16), b2,
                q.astype(jnp.bfloat16), fcb]
    return out


def _prep_seq_params(p):
    out = [p['ti_w'].reshape(72, 48),
           p['bn_g'].reshape(1, 48) * np.float32(1.0 / np.sqrt(1.0 + 1e-5)),
           p['bn_b'].reshape(1, 48)]
    for i in range(3):
        for d in (1, 2, 4):
            out += [p[f'blk{i}_d{d}_w'], p[f'blk{i}_d{d}_b'].reshape(1, 48)]
    out += [p['gp_w'].reshape(1, 1, 48), p['gp_b'].reshape(1, 1),
            p['tr_w'], p['tr_b'].reshape(1, 64),
            p['ln_g'].reshape(1, NTAB), p['ln_b'].reshape(1, NTAB),
            p['tb1_w'], p['tb1_b'].reshape(1, 64),
            p['tb2_w'], p['tb2_b'].reshape(1, 64),
            p['t2t_w'], p['t2t_b'].reshape(1, 64),
            p['fus_w'], p['fus_b'].reshape(1, 96),
            p['sc1_w'], p['sc1_b'].reshape(1, 96),
            p['sc2_w'], p['sc2_b'].reshape(1, NSZ),
            p['so1_w'], p['so1_b'].reshape(1, 48),
            p['so2_w'], p['so2_b'].reshape(1, 6),
            p['sr1_w'], p['sr1_b'].reshape(1, 96),
            p['sr2_w'], p['sr2_b'].reshape(1, 1)]
    return out


def _full_spec(a):
    nd = a.ndim
    return pl.BlockSpec(a.shape, lambda i, _nd=nd: (0,) * _nd)


def kernel(raw_window, norm_window, tabular_x, params):
    enc_consts = _prep_enc_params(params)
    seq_consts = _prep_seq_params(params)

    enc_fn = pl.pallas_call(
        _enc_body,
        grid=(B // BS,),
        in_specs=[
            pl.BlockSpec((BS, SEQ, HW, HW), lambda i: (i, 0, 0, 0)),
            pl.BlockSpec((BS, SEQ, HW, HW), lambda i: (i, 0, 0, 0)),
        ] + [_full_spec(a) for a in enc_consts],
        out_specs=[
            pl.BlockSpec((F, 72), lambda i: (i, 0)),
            pl.BlockSpec((BS, SEQ), lambda i: (i, 0)),
        ],
        out_shape=[
            jax.ShapeDtypeStruct((B * SEQ, 72), jnp.float32),
            jax.ShapeDtypeStruct((B, SEQ), jnp.float32),
        ],
    )
    feat, amp = enc_fn(raw_window, norm_window, *enc_consts)

    n_seq = len(seq_consts)
    seq_fn = pl.pallas_call(
        lambda *refs: _seq_body(refs, n_seq),
        grid=(B // BS2,),
        in_specs=[
            pl.BlockSpec((F2, 72), lambda i: (i, 0)),
            pl.BlockSpec((BS2, SEQ), lambda i: (i, 0)),
            pl.BlockSpec((BS2, NTAB), lambda i: (i, 0)),
        ] + [_full_spec(a) for a in seq_consts],
        out_specs=[
            pl.BlockSpec((BS2, NSZ), lambda i: (i, 0)),
            pl.BlockSpec((BS2, 6), lambda i: (i, 0)),
            pl.BlockSpec((BS2, 1), lambda i: (i, 0)),
            pl.BlockSpec((BS2, NSZ), lambda i: (i, 0)),
        ],
        out_shape=[
            jax.ShapeDtypeStruct((B, NSZ), jnp.float32),
            jax.ShapeDtypeStruct((B, 6), jnp.float32),
            jax.ShapeDtypeStruct((B, 1), jnp.float32),
            jax.ShapeDtypeStruct((B, NSZ), jnp.float32),
        ],
    )
    return tuple(seq_fn(feat, amp, tabular_x, *seq_consts))


# speedup vs baseline: 1.2595x; 1.0689x over previous
import numpy as np
import jax
import jax.numpy as jnp
from jax.experimental import pallas as pl

B = 1024
SEQ = 10
HW = 16
NTAB = 19
NSZ = 7
C = 24
WC = HW * C
TC48 = SEQ * 48

BS = 32
F = BS * SEQ
R = F * HW

BS2 = 256
F2 = BS2 * SEQ

_INV_SQRT2 = float(1.0 / np.sqrt(2.0))

_SEL = np.zeros((HW, 3, HW), np.float32)
for _wi in range(HW):
    for _t in range(3):
        _w = _wi - _t + 1
        if 0 <= _w < HW:
            _SEL[_wi, _t, _w] = 1.0

_DSEL = {}
for _d in (1, 2, 4):
    m = np.zeros((SEQ, SEQ, 3), np.float32)
    for _t in range(SEQ):
        for _tap in range(3):
            _ti = _t + (_tap - 1) * _d
            if 0 <= _ti < SEQ:
                m[_ti, _t, _tap] = 1.0
    _DSEL[_d] = m

_EXP = np.kron(np.eye(SEQ, dtype=np.float32), np.ones((1, 48), np.float32))
_TSUM = np.tile(np.eye(48, dtype=np.float32), (SEQ, 1))


def _dot(a, b):
    return jax.lax.dot_general(a, b, (((1,), (0,)), ((), ())),
                               preferred_element_type=jnp.float32)


def _gelu(v):
    return v * 0.5 * (1.0 + jax.lax.erf(v * _INV_SQRT2))


def _shift_rows(a, off):
    n, c = a.shape
    if off > 0:
        return jnp.concatenate(
            [a[off:], jnp.zeros((off, c), a.dtype)], axis=0)
    if off < 0:
        return jnp.concatenate(
            [jnp.zeros((-off, c), a.dtype), a[:off]], axis=0)
    return a


def _softmax(x):
    m = jnp.max(x, axis=1, keepdims=True)
    e = jnp.exp(x - m)
    return e / jnp.sum(e, axis=1, keepdims=True)


def _conv_stack(a, m_neg, m_pos, ones8):
    return jnp.concatenate(
        [m_neg * _shift_rows(a, -1), a, m_pos * _shift_rows(a, 1), ones8],
        axis=1)


def _encoder(x2b, m_neg, m_pos, ones8, a1, a2, q, fcb):
    r1 = jax.nn.relu(_dot(_conv_stack(x2b, m_neg, m_pos, ones8), a1))
    r1b = r1.astype(jnp.bfloat16)
    r2 = jax.nn.relu(_dot(_conv_stack(r1b, m_neg, m_pos, ones8), a2))
    t = _dot(r2.astype(jnp.bfloat16), q)
    return t.reshape(F, HW, C).sum(axis=1) + fcb


def _enc_body(*refs):
    raw_ref, norm_ref = refs[0], refs[1]
    cw = [r[...] for r in refs[2:14]]
    feat_out, amp_out = refs[14], refs[15]

    raw = raw_ref[...]
    norm = norm_ref[...]
    delta = jnp.concatenate(
        [jnp.zeros_like(norm[:, :1]), norm[:, 1:] - norm[:, :-1]], axis=1)

    hrow = jax.lax.broadcasted_iota(jnp.int32, (R, 1), 0) % HW
    m_neg = (hrow >= 1).astype(jnp.bfloat16)
    m_pos = (hrow <= HW - 2).astype(jnp.bfloat16)
    ones8 = jnp.ones((R, 8), jnp.bfloat16)

    outs = []
    for k, x4 in enumerate((raw, norm, delta)):
        x2b = x4.reshape(R, HW).astype(jnp.bfloat16)
        outs.append(_encoder(x2b, m_neg, m_pos, ones8, *cw[4 * k:4 * k + 4]))
    feat_out[...] = jnp.concatenate(outs, axis=1)
    amp_out[...] = raw.sum(axis=3).sum(axis=2) * np.float32(1.0 / (HW * HW))


def _seq_body(refs, n_params):
    feat_ref, amp_ref, tab_ref = refs[0], refs[1], refs[2]
    c = [r[...] for r in refs[3:3 + n_params]]
    ol, oo, org, oprob = refs[3 + n_params:]

    it = iter(c)

    def nxt():
        return next(it)

    ti_w = nxt()
    bn_sc, bn_b = nxt(), nxt()
    wacc = [(nxt(), nxt()) for _ in range(3)]
    gp_m, gp_b = nxt(), nxt()
    exp_m, tsum_m = nxt(), nxt()
    tr_w, tr_b = nxt(), nxt()
    ln_g, ln_b = nxt(), nxt()
    tb1_w, tb1_b = nxt(), nxt()
    tb2_w, tb2_b = nxt(), nxt()
    t2t_w, t2t_b = nxt(), nxt()
    fus_w, fus_b = nxt(), nxt()
    sc1_w, sc1_b = nxt(), nxt()
    sc2_w, sc2_b = nxt(), nxt()
    so1_w, so1_b = nxt(), nxt()
    so2_w, so2_b = nxt(), nxt()
    sr1_w, sr1_b = nxt(), nxt()
    sr2_w, sr2_b = nxt(), nxt()

    feat = feat_ref[...]
    amp = amp_ref[...]
    tab = tab_ref[...]

    s = jax.nn.relu(_dot(feat, ti_w) * bn_sc + bn_b)
    s3 = s.reshape(BS2, SEQ, 48)
    st = jnp.concatenate([s3[:, t] for t in range(SEQ)], axis=1)

    for i in range(3):
        w, bias = wacc[i]
        st = jax.nn.relu(st + _dot(st, w) + bias)

    scores = _dot(st, gp_m) + gp_b
    attn = _softmax(scores)

    thr = amp.mean(axis=1, keepdims=True)
    m_hi = (amp >= thr).astype(jnp.float32)
    m_lo = 1.0 - m_hi

    gfeat = _dot(st * _dot(attn, exp_m), tsum_m)
    ph = (_dot(st * _dot(m_hi, exp_m), tsum_m) /
          (jnp.sum(m_hi, axis=1, keepdims=True) + 1e-6))
    plo = (_dot(st * _dot(m_lo, exp_m), tsum_m) /
           (jnp.sum(m_lo, axis=1, keepdims=True) + 1e-6))

    fused = jnp.concatenate([gfeat, ph, plo], axis=1)
    trunk = jax.nn.relu(_dot(fused, tr_w) + tr_b)

    mu = tab.mean(axis=1, keepdims=True)
    xc = tab - mu
    var = (xc * xc).mean(axis=1, keepdims=True)
    tn = xc / jnp.sqrt(var + 1e-5) * ln_g + ln_b
    tf = _gelu(_dot(tn, tb1_w) + tb1_b)
    tf = _gelu(_dot(tf, tb2_w) + tb2_b)
    tp = _dot(tf, t2t_w) + t2t_b

    hyb = jax.nn.relu(
        _dot(jnp.concatenate([trunk, tp, trunk * tp], axis=1), fus_w) + fus_b)
    sl = _dot(jax.nn.relu(_dot(hyb, sc1_w) + sc1_b), sc2_w) + sc2_b
    so = _dot(jax.nn.relu(_dot(hyb, so1_w) + so1_b), so2_w) + so2_b
    sp = _softmax(sl)
    vals = (jax.lax.broadcasted_iota(jnp.int32, (1, NSZ), 1).astype(
        jnp.float32) * np.float32(1.0 / (NSZ - 1)))
    expected = jnp.sum(sp * vals, axis=1, keepdims=True)
    res = 0.35 * jnp.tanh(
        _dot(jax.nn.relu(_dot(hyb, sr1_w) + sr1_b), sr2_w) + sr2_b)
    rg = jnp.clip(expected + res, 0.0, 1.0)

    ol[...] = sl
    oo[...] = so
    org[...] = rg
    oprob[...] = sp


def _prep_enc_params(p):
    sel = jnp.asarray(_SEL)
    out = []
    for enc in ('amp', 'shp', 'dlt'):
        w1 = p[enc + '_c1w'].reshape(3, 3, C)
        a1 = jnp.einsum('itw,otc->oiwc', sel, w1).reshape(3 * HW, WC)
        b1 = jnp.tile(p[enc + '_c1b'], HW).reshape(1, WC)
        a1 = jnp.concatenate(
            [a1, b1, jnp.zeros((7, WC), jnp.float32)], axis=0)
        w2 = p[enc + '_c2w']
        a2 = jnp.einsum('itw,otcd->oicwd', sel, w2).reshape(3 * WC, WC)
        b2 = jnp.tile(p[enc + '_c2b'], HW).reshape(1, WC)
        a2 = jnp.concatenate(
            [a2, b2, jnp.zeros((7, WC), jnp.float32)], axis=0)
        q = jnp.tile(p[enc + '_fcw'] * np.float32(1.0 / (HW * HW)), (HW, 1))
        fcb = p[enc + '_fcb'].reshape(1, C)
        out += [a1.astype(jnp.bfloat16), a2.astype(jnp.bfloat16),
                q.astype(jnp.bfloat16), fcb]
    return out


def _prep_seq_params(p):
    out = [p['ti_w'].reshape(72, 48),
           p['bn_g'].reshape(1, 48) * np.float32(1.0 / np.sqrt(1.0 + 1e-5)),
           p['bn_b'].reshape(1, 48)]
    for i in range(3):
        wsum = None
        bsum = None
        for d in (1, 2, 4):
            dsel = jnp.asarray(_DSEL[d])
            w = jnp.einsum('utp,pcd->uctd',
                           dsel, p[f'blk{i}_d{d}_w']).reshape(TC48, TC48)
            wsum = w if wsum is None else wsum + w
            bd = p[f'blk{i}_d{d}_b']
            bsum = bd if bsum is None else bsum + bd
        out += [wsum, jnp.tile(bsum, SEQ).reshape(1, TC48)]
    gp_m = jnp.kron(jnp.eye(SEQ, dtype=jnp.float32),
                    p['gp_w'].reshape(48, 1))
    out += [gp_m, p['gp_b'].reshape(1, 1),
            jnp.asarray(_EXP), jnp.asarray(_TSUM),
            p['tr_w'], p['tr_b'].reshape(1, 64),
            p['ln_g'].reshape(1, NTAB), p['ln_b'].reshape(1, NTAB),
            p['tb1_w'], p['tb1_b'].reshape(1, 64),
            p['tb2_w'], p['tb2_b'].reshape(1, 64),
            p['t2t_w'], p['t2t_b'].reshape(1, 64),
            p['fus_w'], p['fus_b'].reshape(1, 96),
            p['sc1_w'], p['sc1_b'].reshape(1, 96),
            p['sc2_w'], p['sc2_b'].reshape(1, NSZ),
            p['so1_w'], p['so1_b'].reshape(1, 48),
            p['so2_w'], p['so2_b'].reshape(1, 6),
            p['sr1_w'], p['sr1_b'].reshape(1, 96),
            p['sr2_w'], p['sr2_b'].reshape(1, 1)]
    return out


def _full_spec(a):
    nd = a.ndim
    return pl.BlockSpec(a.shape, lambda i, _nd=nd: (0,) * _nd)


def kernel(raw_window, norm_window, tabular_x, params):
    enc_consts = _prep_enc_params(params)
    seq_consts = _prep_seq_params(params)

    enc_fn = pl.pallas_call(
        _enc_body,
        grid=(B // BS,),
        in_specs=[
            pl.BlockSpec((BS, SEQ, HW, HW), lambda i: (i, 0, 0, 0)),
            pl.BlockSpec((BS, SEQ, HW, HW), lambda i: (i, 0, 0, 0)),
        ] + [_full_spec(a) for a in enc_consts],
        out_specs=[
            pl.BlockSpec((F, 72), lambda i: (i, 0)),
            pl.BlockSpec((BS, SEQ), lambda i: (i, 0)),
        ],
        out_shape=[
            jax.ShapeDtypeStruct((B * SEQ, 72), jnp.float32),
            jax.ShapeDtypeStruct((B, SEQ), jnp.float32),
        ],
    )
    feat, amp = enc_fn(raw_window, norm_window, *enc_consts)

    n_seq = len(seq_consts)
    seq_fn = pl.pallas_call(
        lambda *refs: _seq_body(refs, n_seq),
        grid=(B // BS2,),
        in_specs=[
            pl.BlockSpec((F2, 72), lambda i: (i, 0)),
            pl.BlockSpec((BS2, SEQ), lambda i: (i, 0)),
            pl.BlockSpec((BS2, NTAB), lambda i: (i, 0)),
        ] + [_full_spec(a) for a in seq_consts],
        out_specs=[
            pl.BlockSpec((BS2, NSZ), lambda i: (i, 0)),
            pl.BlockSpec((BS2, 6), lambda i: (i, 0)),
            pl.BlockSpec((BS2, 1), lambda i: (i, 0)),
            pl.BlockSpec((BS2, NSZ), lambda i: (i, 0)),
        ],
        out_shape=[
            jax.ShapeDtypeStruct((B, NSZ), jnp.float32),
            jax.ShapeDtypeStruct((B, 6), jnp.float32),
            jax.ShapeDtypeStruct((B, 1), jnp.float32),
            jax.ShapeDtypeStruct((B, NSZ), jnp.float32),
        ],
    )
    return tuple(seq_fn(feat, amp, tabular_x, *seq_consts))
